# baseline (speedup 1.0000x reference)
import functools

import jax
import jax.numpy as jnp
from jax import lax
from jax.experimental import pallas as pl
from jax.experimental.pallas import tpu as pltpu
from jax.experimental.pallas import tpu_sc as plsc

N = 10000
E = 320000
S = 128
TW = 256
XP = 16
NRBF = 18
RBP = 32
NC = 2
NS = 16
NW = NC * NS
NSL = 2

BE = 800
BN = 2000


def _chunk_of(n):
    for g in range(128, 0, -8):
        if n % g == 0 and (n // g) % 2 == 1:
            return g
    raise ValueError(n)


def _make_gather(esl):
    epw = esl // NW
    gc = _chunk_of(epw)
    n = epw // gc
    npair = (n - 1) // 2

    def body(t_hbm, src_hbm, dst_hbm, ts_out, td_out,
             srcv0, dstv0, tsv0, tdv0, srcv1, dstv1, tsv1, tdv1,
             gsem0, gsem1):
        c = lax.axis_index("c")
        s = lax.axis_index("s")
        wid = s * NC + c
        base0 = wid * epw

        def load_idx(j, sv, dv):
            pltpu.sync_copy(src_hbm.at[pl.ds(base0 + j * gc, gc)], sv)
            pltpu.sync_copy(dst_hbm.at[pl.ds(base0 + j * gc, gc)], dv)

        def fire(sv, dv, tsv, tdv, sem):
            pltpu.async_copy(t_hbm.at[sv], tsv, sem)
            return pltpu.async_copy(t_hbm.at[dv], tdv, sem)

        def writeback(j, tsv, tdv):
            pltpu.sync_copy(tsv, ts_out.at[pl.ds(base0 + j * gc, gc)])
            pltpu.sync_copy(tdv, td_out.at[pl.ds(base0 + j * gc, gc)])

        load_idx(0, srcv0, dstv0)
        fire(srcv0, dstv0, tsv0, tdv0, gsem0)

        def pair(j2, carry):
            ja = 2 * j2
            jb = ja + 1
            jc = ja + 2
            load_idx(jb, srcv1, dstv1)
            cb = fire(srcv1, dstv1, tsv1, tdv1, gsem1)
            ca = pltpu.make_async_copy(t_hbm.at[srcv0], tsv0, gsem0)
            ca.wait()
            ca.wait()
            writeback(ja, tsv0, tdv0)
            load_idx(jc, srcv0, dstv0)
            fire(srcv0, dstv0, tsv0, tdv0, gsem0)
            cb.wait()
            cb.wait()
            writeback(jb, tsv1, tdv1)
            return carry

        lax.fori_loop(0, npair, pair, 0)

        ce = pltpu.make_async_copy(t_hbm.at[srcv0], tsv0, gsem0)
        ce.wait()
        ce.wait()
        writeback(n - 1, tsv0, tdv0)

    mesh = plsc.VectorSubcoreMesh(core_axis_name="c", subcore_axis_name="s")
    return pl.kernel(
        body,
        out_type=(
            jax.ShapeDtypeStruct((esl, TW // 2), jnp.int32),
            jax.ShapeDtypeStruct((esl, TW // 2), jnp.int32),
        ),
        mesh=mesh,
        scratch_types=[
            pltpu.VMEM((gc,), jnp.int32),
            pltpu.VMEM((gc,), jnp.int32),
            pltpu.VMEM((gc, TW // 2), jnp.int32),
            pltpu.VMEM((gc, TW // 2), jnp.int32),
            pltpu.VMEM((gc,), jnp.int32),
            pltpu.VMEM((gc,), jnp.int32),
            pltpu.VMEM((gc, TW // 2), jnp.int32),
            pltpu.VMEM((gc, TW // 2), jnp.int32),
            pltpu.SemaphoreType.DMA,
            pltpu.SemaphoreType.DMA,
        ],
    )


def _make_scatter(esl):
    ept = esl // NS
    gc = _chunk_of(ept)
    n = ept // gc
    npair = (n - 1) // 2

    def body(dh_hbm, dxp_hbm, dst_hbm, zeros_hbm,
             aggh_out, aggx_out,
             dstv0, rowv0, dstv1, rowv1, acc, rsem0, rsem1):
        c = lax.axis_index("c")
        s = lax.axis_index("s")
        base0 = s * ept

        @pl.when(s == 0)
        def _():
            pltpu.sync_copy(zeros_hbm, acc)

        plsc.subcore_barrier()

        def run(msg_hbm):
            def load(j, dv, rv, sem):
                pltpu.sync_copy(dst_hbm.at[pl.ds(base0 + j * gc, gc)], dv)
                return pltpu.async_copy(
                    msg_hbm.at[pl.ds(base0 + j * gc, gc)], rv, sem)

            def drain0():
                pltpu.make_async_copy(
                    msg_hbm.at[pl.ds(base0, gc)], rowv0, rsem0).wait()

            load(0, dstv0, rowv0, rsem0)

            def pair(j2, carry):
                jb = 2 * j2 + 1
                jc = jb + 1
                cb = load(jb, dstv1, rowv1, rsem1)
                drain0()
                pltpu.sync_copy(rowv0, acc.at[dstv0], add=True)
                load(jc, dstv0, rowv0, rsem0)
                cb.wait()
                pltpu.sync_copy(rowv1, acc.at[dstv1], add=True)
                return carry

            lax.fori_loop(0, npair, pair, 0)
            drain0()
            pltpu.sync_copy(rowv0, acc.at[dstv0], add=True)

        @pl.when(c == 0)
        def _():
            run(dh_hbm)

        @pl.when(c == 1)
        def _():
            run(dxp_hbm)

        plsc.subcore_barrier()

        @pl.when((c == 0) & (s == 0))
        def _():
            pltpu.sync_copy(acc, aggh_out)

        @pl.when((c == 1) & (s == 0))
        def _():
            pltpu.sync_copy(acc, aggx_out)

    mesh = plsc.VectorSubcoreMesh(core_axis_name="c", subcore_axis_name="s")
    return pl.kernel(
        body,
        out_type=(
            jax.ShapeDtypeStruct((N, S), jnp.float32),
            jax.ShapeDtypeStruct((N, S), jnp.float32),
        ),
        mesh=mesh,
        scratch_types=[
            pltpu.VMEM((gc,), jnp.int32),
            pltpu.VMEM((gc, S), jnp.float32),
            pltpu.VMEM((gc,), jnp.int32),
            pltpu.VMEM((gc, S), jnp.float32),
            pltpu.VMEM_SHARED((N, S), jnp.float32),
            pltpu.SemaphoreType.DMA,
            pltpu.SemaphoreType.DMA,
        ],
    )


def _silu(z):
    return z * jax.nn.sigmoid(z)


def _mlp_body(ts, td, es, cent, wiv,
              w1ae, w1ao, w1be, w1bo, w1c, w1d, b1, w2, b2, w3, b3,
              wg1, bg1, wg2, bg2, wh, bh, wx1, bx1, wx2, bx2,
              dh_out, dx_out):
    hm = jnp.int32(-65536)

    def planes(w32):
        ev = lax.bitcast_convert_type(lax.shift_left(w32, 16), jnp.float32)
        od = lax.bitcast_convert_type(jnp.bitwise_and(w32, hm), jnp.float32)
        return ev, od

    se, so = planes(ts[...])
    de, do_ = planes(td[...])
    hse, hso = se[:, :S // 2], so[:, :S // 2]
    hde, hdo = de[:, :S // 2], do_[:, :S // 2]
    rxe = se[:, S // 2:] - de[:, S // 2:]
    rxo = so[:, S // 2:] - do_[:, S // 2:]
    d2 = (jnp.sum(rxe * rxe, axis=1, keepdims=True)
          + jnp.sum(rxo * rxo, axis=1, keepdims=True))
    dist = jnp.sqrt(d2)
    t = (dist - cent[...]) * wiv[0, 0]
    rbf = jnp.exp(-(t * t))

    bf = jnp.bfloat16

    def dot(a_, b_):
        return jnp.dot(a_.astype(bf), b_.astype(bf),
                       preferred_element_type=jnp.float32)

    m = (dot(hse, w1ae[...]) + dot(hso, w1ao[...])
         + dot(hde, w1be[...]) + dot(hdo, w1bo[...])
         + dot(rbf, w1c[...]) + dot(es[...], w1d[...]) + b1[...])
    m = _silu(m)
    m = _silu(dot(m, w2[...]) + b2[...])
    m = _silu(dot(m, w3[...]) + b3[...])
    a = jax.nn.relu(dot(m, wg1[...]) + bg1[...])
    g = jax.nn.sigmoid(jnp.sum(a * wg2[...], axis=1, keepdims=True) + bg2[0, 0])
    m = m * g
    dh_out[...] = _silu(dot(m, wh[...]) + bh[...])
    cx = _silu(dot(m, wx1[...]) + bx1[...])
    coeff = (jnp.sum(cx * wx2[...], axis=1, keepdims=True) + bx2[0, 0]) * 0.08
    dx_out[...] = jnp.concatenate([rxe, rxo], axis=1) * coeff


def _full(shape):
    return pl.BlockSpec(shape, lambda i: (0,) * len(shape))


def _mlp_call(ts, td, es, cent, wiv, wts):
    esl = ts.shape[0]
    in_specs = [
        pl.BlockSpec((BE, TW // 2), lambda i: (i, 0)),
        pl.BlockSpec((BE, TW // 2), lambda i: (i, 0)),
        pl.BlockSpec((BE, 16), lambda i: (i, 0)),
        _full(cent.shape), _full(wiv.shape),
    ] + [_full(w.shape) for w in wts]
    return pl.pallas_call(
        _mlp_body,
        grid=(esl // BE,),
        in_specs=in_specs,
        out_specs=[
            pl.BlockSpec((BE, S), lambda i: (i, 0)),
            pl.BlockSpec((BE, S), lambda i: (i, 0)),
        ],
        out_shape=[
            jax.ShapeDtypeStruct((esl, S), jnp.float32),
            jax.ShapeDtypeStruct((esl, S), jnp.float32),
        ],
    )(ts, td, es, cent, wiv, *wts)


def _fin_body(h, x16, lng, lnb, alpha, *refs):
    n_agg = (len(refs) - 2) // 2
    agghs = refs[:n_agg]
    aggxs = refs[n_agg:2 * n_agg]
    h_out, x_out = refs[2 * n_agg:]
    sa = jax.nn.sigmoid(alpha[0, 0])
    aggh = agghs[0][...]
    aggx = aggxs[0][...]
    for k in range(1, n_agg):
        aggh = aggh + agghs[k][...]
        aggx = aggx + aggxs[k][...]
    pre = h[...] + sa * aggh
    mu = jnp.mean(pre, axis=1, keepdims=True)
    cent = pre - mu
    var = jnp.mean(cent * cent, axis=1, keepdims=True)
    h_out[...] = cent * lax.rsqrt(var + 1e-5) * lng[...] + lnb[...]
    xdec = jnp.concatenate(
        [aggx[:, 0:1], aggx[:, 64:65], aggx[:, 1:2],
         jnp.zeros((aggx.shape[0], XP - 3), jnp.float32)], axis=1)
    x_out[...] = x16[...] + xdec


def _fin_call(h, x16, lng, lnb, alpha, agghs, aggxs):
    nsb = pl.BlockSpec((BN, S), lambda i: (i, 0))
    xsb = pl.BlockSpec((BN, XP), lambda i: (i, 0))
    return pl.pallas_call(
        _fin_body,
        grid=(N // BN,),
        in_specs=[
            nsb, xsb,
            _full(lng.shape), _full(lnb.shape), _full(alpha.shape),
        ] + [nsb] * (len(agghs) + len(aggxs)),
        out_specs=[nsb, xsb],
        out_shape=[
            jax.ShapeDtypeStruct((N, S), jnp.float32),
            jax.ShapeDtypeStruct((N, XP), jnp.float32),
        ],
    )(h, x16, lng, lnb, alpha, *agghs, *aggxs)


def kernel(h, x, edge_index, e_s, params, centers, widths):
    src = edge_index[0].astype(jnp.int32)
    dst = edge_index[1].astype(jnp.int32)
    xf = x.astype(jnp.float32)
    x16 = jnp.pad(xf, ((0, 0), (0, XP - 3)))
    T = lax.bitcast_convert_type(
        jnp.concatenate([h, xf, jnp.zeros((N, TW - S - 3), jnp.float32)],
                        axis=1).astype(jnp.bfloat16).reshape(N, TW // 2, 2),
        jnp.int32)

    W1 = params['W1']
    w1ae = W1[:, 0:S:2].T
    w1ao = W1[:, 1:S:2].T
    w1be = W1[:, S:2 * S:2].T
    w1bo = W1[:, S + 1:2 * S:2].T
    w1c = jnp.pad(W1[:, 2 * S:2 * S + NRBF].T, ((0, RBP - NRBF), (0, 0)))
    w1d = W1[:, 2 * S + NRBF:].T
    b1 = params['b1'][None, :]
    w2 = params['W2'].T
    b2 = params['b2'][None, :]
    w3 = params['W3'].T
    b3 = params['b3'][None, :]
    wg1 = params['Wg1'].T
    bg1 = params['bg1'][None, :]
    wg2 = params['Wg2']
    bg2 = params['bg2'][None, :]
    wh = params['Wh'].T
    bh = params['bh'][None, :]
    wx1 = params['Wx1'].T
    bx1 = params['bx1'][None, :]
    wx2 = params['Wx2']
    bx2 = params['bx2'][None, :]
    cent = jnp.pad(centers[None, :], ((0, 0), (0, RBP - NRBF)))
    wiv = (1.0 / (widths + 1e-8)).reshape(1, 1)
    alpha = params['alpha'].reshape(1, 1)
    lng = params['ln_g'][None, :]
    lnb = params['ln_b'][None, :]

    wts = [w1ae, w1ao, w1be, w1bo, w1c, w1d, b1, w2, b2, w3, b3,
           wg1, bg1, wg2, bg2, wh, bh, wx1, bx1, wx2, bx2]
    zeros = jnp.zeros((N, S), jnp.float32)

    esl = E // NSL
    gather_fn = _make_gather(esl)
    scatter_fn = _make_scatter(esl)
    agghs, aggxs = [], []
    for k in range(NSL):
        sl = slice(k * esl, (k + 1) * esl)
        ts, td = gather_fn(T, src[sl], dst[sl])
        dh, dxp = _mlp_call(ts, td, e_s[sl], cent, wiv, wts)
        aggh_k, aggx_k = scatter_fn(dh, dxp, dst[sl], zeros)
        agghs.append(aggh_k)
        aggxs.append(aggx_k)

    h_new, x_new16 = _fin_call(h, x16, lng, lnb, alpha, agghs, aggxs)
    return (h_new, x_new16[:, :3])


# speedup vs baseline: 1.2415x; 1.2415x over previous
import functools

import jax
import jax.numpy as jnp
from jax import lax
from jax.experimental import pallas as pl
from jax.experimental.pallas import tpu as pltpu
from jax.experimental.pallas import tpu_sc as plsc

N = 10000
E = 320000
S = 128
TW = 256
XP = 16
NRBF = 18
RBP = 32
NC = 2
NS = 16
NW = NC * NS
NSL = 2

BE = 800
BN = 2000


def _chunk_of(n):
    for g in range(128, 0, -8):
        if n % g == 0 and (n // g) % 2 == 1:
            return g
    raise ValueError(n)


def _make_gather(esl):
    epw = esl // NW
    gc = _chunk_of(epw)
    n = epw // gc
    npair = (n - 1) // 2

    def body(t_hbm, src3_hbm, dst3_hbm, ts_out, td_out,
             srcall, dstall, tsv0, tdv0, tsv1, tdv1, gsem0, gsem1):
        c = lax.axis_index("c")
        s = lax.axis_index("s")
        wid = s * NC + c
        base0 = wid * epw

        pltpu.sync_copy(src3_hbm.at[wid], srcall)
        pltpu.sync_copy(dst3_hbm.at[wid], dstall)

        def fire(j, tsv, tdv, sem):
            pltpu.async_copy(t_hbm.at[srcall.at[j]], tsv, sem)
            return pltpu.async_copy(t_hbm.at[dstall.at[j]], tdv, sem)

        def drain(tsv, sem):
            d = pltpu.make_async_copy(t_hbm.at[srcall.at[0]], tsv, sem)
            d.wait()
            d.wait()

        def writeback(j, tsv, tdv):
            pltpu.sync_copy(tsv, ts_out.at[pl.ds(base0 + j * gc, gc)])
            pltpu.sync_copy(tdv, td_out.at[pl.ds(base0 + j * gc, gc)])

        fire(0, tsv0, tdv0, gsem0)

        def pair(j2, carry):
            ja = 2 * j2
            jb = ja + 1
            jc = ja + 2
            fire(jb, tsv1, tdv1, gsem1)
            drain(tsv0, gsem0)
            writeback(ja, tsv0, tdv0)
            fire(jc, tsv0, tdv0, gsem0)
            drain(tsv1, gsem1)
            writeback(jb, tsv1, tdv1)
            return carry

        lax.fori_loop(0, npair, pair, 0)

        drain(tsv0, gsem0)
        writeback(n - 1, tsv0, tdv0)

    mesh = plsc.VectorSubcoreMesh(core_axis_name="c", subcore_axis_name="s")
    return pl.kernel(
        body,
        out_type=(
            jax.ShapeDtypeStruct((esl, TW), jnp.float32),
            jax.ShapeDtypeStruct((esl, TW), jnp.float32),
        ),
        mesh=mesh,
        scratch_types=[
            pltpu.VMEM((n, gc), jnp.int32),
            pltpu.VMEM((n, gc), jnp.int32),
            pltpu.VMEM((gc, TW), jnp.float32),
            pltpu.VMEM((gc, TW), jnp.float32),
            pltpu.VMEM((gc, TW), jnp.float32),
            pltpu.VMEM((gc, TW), jnp.float32),
            pltpu.SemaphoreType.DMA,
            pltpu.SemaphoreType.DMA,
        ],
    )


def _make_scatter(esl):
    ept = esl // NS
    gc = _chunk_of(ept)
    n = ept // gc
    npair = (n - 1) // 2

    def body(dh_hbm, dxp_hbm, dst3_hbm, zeros_hbm,
             aggh_out, aggx_out,
             dstall, rowv0, rowv1, acc, rsem0, rsem1):
        c = lax.axis_index("c")
        s = lax.axis_index("s")
        base0 = s * ept

        @pl.when(s == 0)
        def _():
            pltpu.sync_copy(zeros_hbm, acc)

        pltpu.sync_copy(dst3_hbm.at[s], dstall)
        plsc.subcore_barrier()

        def run(msg_hbm):
            def load(j, rv, sem):
                return pltpu.async_copy(
                    msg_hbm.at[pl.ds(base0 + j * gc, gc)], rv, sem)

            def drain0():
                pltpu.make_async_copy(
                    msg_hbm.at[pl.ds(base0, gc)], rowv0, rsem0).wait()

            load(0, rowv0, rsem0)

            def pair(j2, carry):
                ja = 2 * j2
                jb = ja + 1
                jc = ja + 2
                cb = load(jb, rowv1, rsem1)
                drain0()
                pltpu.sync_copy(rowv0, acc.at[dstall.at[ja]], add=True)
                load(jc, rowv0, rsem0)
                cb.wait()
                pltpu.sync_copy(rowv1, acc.at[dstall.at[jb]], add=True)
                return carry

            lax.fori_loop(0, npair, pair, 0)
            drain0()
            pltpu.sync_copy(rowv0, acc.at[dstall.at[n - 1]], add=True)

        @pl.when(c == 0)
        def _():
            run(dh_hbm)

        @pl.when(c == 1)
        def _():
            run(dxp_hbm)

        plsc.subcore_barrier()

        @pl.when((c == 0) & (s == 0))
        def _():
            pltpu.sync_copy(acc, aggh_out)

        @pl.when((c == 1) & (s == 0))
        def _():
            pltpu.sync_copy(acc, aggx_out)

    mesh = plsc.VectorSubcoreMesh(core_axis_name="c", subcore_axis_name="s")
    return pl.kernel(
        body,
        out_type=(
            jax.ShapeDtypeStruct((N, S), jnp.float32),
            jax.ShapeDtypeStruct((N, S), jnp.float32),
        ),
        mesh=mesh,
        scratch_types=[
            pltpu.VMEM((n, gc), jnp.int32),
            pltpu.VMEM((gc, S), jnp.float32),
            pltpu.VMEM((gc, S), jnp.float32),
            pltpu.VMEM_SHARED((N, S), jnp.float32),
            pltpu.SemaphoreType.DMA,
            pltpu.SemaphoreType.DMA,
        ],
    )


def _silu(z):
    return z * jax.nn.sigmoid(z)


def _mlp_body(ts, td, es, cent, wiv,
              w1a, w1b, w1c, w1d, b1, w2, b2, w3, b3,
              wg1, bg1, wg2, bg2, wh, bh, wx1, bx1, wx2, bx2,
              dh_out, dx_out):
    hs = ts[:, :S]
    hd = td[:, :S]
    r = ts[:, S:] - td[:, S:]
    d2 = jnp.sum(r * r, axis=1, keepdims=True)
    dist = jnp.sqrt(d2)
    t = (dist - cent[...]) * wiv[0, 0]
    rbf = jnp.exp(-(t * t))

    dot = functools.partial(jnp.dot, preferred_element_type=jnp.float32)
    m = (dot(hs, w1a[...]) + dot(hd, w1b[...])
         + dot(rbf, w1c[...]) + dot(es[...], w1d[...]) + b1[...])
    m = _silu(m)
    m = _silu(dot(m, w2[...]) + b2[...])
    m = _silu(dot(m, w3[...]) + b3[...])
    a = jax.nn.relu(dot(m, wg1[...]) + bg1[...])
    g = jax.nn.sigmoid(jnp.sum(a * wg2[...], axis=1, keepdims=True) + bg2[0, 0])
    m = m * g
    dh_out[...] = _silu(dot(m, wh[...]) + bh[...])
    cx = _silu(dot(m, wx1[...]) + bx1[...])
    coeff = (jnp.sum(cx * wx2[...], axis=1, keepdims=True) + bx2[0, 0]) * 0.08
    dx_out[...] = r * coeff


def _full(shape):
    return pl.BlockSpec(shape, lambda i: (0,) * len(shape))


def _mlp_call(ts, td, es, cent, wiv, wts):
    esl = ts.shape[0]
    in_specs = [
        pl.BlockSpec((BE, TW), lambda i: (i, 0)),
        pl.BlockSpec((BE, TW), lambda i: (i, 0)),
        pl.BlockSpec((BE, 16), lambda i: (i, 0)),
        _full(cent.shape), _full(wiv.shape),
    ] + [_full(w.shape) for w in wts]
    return pl.pallas_call(
        _mlp_body,
        grid=(esl // BE,),
        in_specs=in_specs,
        out_specs=[
            pl.BlockSpec((BE, S), lambda i: (i, 0)),
            pl.BlockSpec((BE, S), lambda i: (i, 0)),
        ],
        out_shape=[
            jax.ShapeDtypeStruct((esl, S), jnp.float32),
            jax.ShapeDtypeStruct((esl, S), jnp.float32),
        ],
    )(ts, td, es, cent, wiv, *wts)


def _fin_body(h, x16, lng, lnb, alpha, *refs):
    n_agg = (len(refs) - 2) // 2
    agghs = refs[:n_agg]
    aggxs = refs[n_agg:2 * n_agg]
    h_out, x_out = refs[2 * n_agg:]
    sa = jax.nn.sigmoid(alpha[0, 0])
    aggh = agghs[0][...]
    aggx = aggxs[0][...]
    for k in range(1, n_agg):
        aggh = aggh + agghs[k][...]
        aggx = aggx + aggxs[k][...]
    pre = h[...] + sa * aggh
    mu = jnp.mean(pre, axis=1, keepdims=True)
    cent = pre - mu
    var = jnp.mean(cent * cent, axis=1, keepdims=True)
    h_out[...] = cent * lax.rsqrt(var + 1e-5) * lng[...] + lnb[...]
    x_out[...] = x16[...] + aggx[:, :XP]


def _fin_call(h, x16, lng, lnb, alpha, agghs, aggxs):
    nsb = pl.BlockSpec((BN, S), lambda i: (i, 0))
    xsb = pl.BlockSpec((BN, XP), lambda i: (i, 0))
    return pl.pallas_call(
        _fin_body,
        grid=(N // BN,),
        in_specs=[
            nsb, xsb,
            _full(lng.shape), _full(lnb.shape), _full(alpha.shape),
        ] + [nsb] * (len(agghs) + len(aggxs)),
        out_specs=[nsb, xsb],
        out_shape=[
            jax.ShapeDtypeStruct((N, S), jnp.float32),
            jax.ShapeDtypeStruct((N, XP), jnp.float32),
        ],
    )(h, x16, lng, lnb, alpha, *agghs, *aggxs)


def kernel(h, x, edge_index, e_s, params, centers, widths):
    src = edge_index[0].astype(jnp.int32)
    dst = edge_index[1].astype(jnp.int32)
    xf = x.astype(jnp.float32)
    x16 = jnp.pad(xf, ((0, 0), (0, XP - 3)))
    T = jnp.concatenate([h, xf, jnp.zeros((N, TW - S - 3), jnp.float32)], axis=1)

    W1 = params['W1']
    w1a = W1[:, :S].T
    w1b = W1[:, S:2 * S].T
    w1c = jnp.pad(W1[:, 2 * S:2 * S + NRBF].T, ((0, RBP - NRBF), (0, 0)))
    w1d = W1[:, 2 * S + NRBF:].T
    b1 = params['b1'][None, :]
    w2 = params['W2'].T
    b2 = params['b2'][None, :]
    w3 = params['W3'].T
    b3 = params['b3'][None, :]
    wg1 = params['Wg1'].T
    bg1 = params['bg1'][None, :]
    wg2 = params['Wg2']
    bg2 = params['bg2'][None, :]
    wh = params['Wh'].T
    bh = params['bh'][None, :]
    wx1 = params['Wx1'].T
    bx1 = params['bx1'][None, :]
    wx2 = params['Wx2']
    bx2 = params['bx2'][None, :]
    cent = jnp.pad(centers[None, :], ((0, 0), (0, RBP - NRBF)))
    wiv = (1.0 / (widths + 1e-8)).reshape(1, 1)
    alpha = params['alpha'].reshape(1, 1)
    lng = params['ln_g'][None, :]
    lnb = params['ln_b'][None, :]

    wts = [w1a, w1b, w1c, w1d, b1, w2, b2, w3, b3,
           wg1, bg1, wg2, bg2, wh, bh, wx1, bx1, wx2, bx2]
    zeros = jnp.zeros((N, S), jnp.float32)

    esl = E // NSL
    epw = esl // NW
    gcg = _chunk_of(epw)
    ept = esl // NS
    gcs_ = _chunk_of(ept)
    gather_fn = _make_gather(esl)
    scatter_fn = _make_scatter(esl)
    agghs, aggxs = [], []
    for k in range(NSL):
        sl = slice(k * esl, (k + 1) * esl)
        src3 = src[sl].reshape(NW, epw // gcg, gcg)
        dst3 = dst[sl].reshape(NW, epw // gcg, gcg)
        dst3s = dst[sl].reshape(NS, ept // gcs_, gcs_)
        ts, td = gather_fn(T, src3, dst3)
        dh, dxp = _mlp_call(ts, td, e_s[sl], cent, wiv, wts)
        aggh_k, aggx_k = scatter_fn(dh, dxp, dst3s, zeros)
        agghs.append(aggh_k)
        aggxs.append(aggx_k)

    h_new, x_new16 = _fin_call(h, x16, lng, lnb, alpha, agghs, aggxs)
    return (h_new, x_new16[:, :3])


# speedup vs baseline: 1.3595x; 1.0950x over previous
import functools

import jax
import jax.numpy as jnp
from jax import lax
from jax.experimental import pallas as pl
from jax.experimental.pallas import tpu as pltpu
from jax.experimental.pallas import tpu_sc as plsc

N = 10000
E = 320000
S = 128
TW = 256
XP = 16
NRBF = 18
RBP = 32
NC = 2
NS = 16
NW = NC * NS
NSL = 2

BE = 1600
BN = 2000


def _chunk_of(n):
    for g in range(128, 0, -8):
        if n % g == 0 and (n // g) % 2 == 1:
            return g
    raise ValueError(n)


def _make_gather(esl):
    epw = esl // NW
    gc = _chunk_of(epw)
    n = epw // gc
    npair = (n - 1) // 2

    def body(t_hbm, src3_hbm, dst3_hbm, ts_out, td_out,
             srcall, dstall, tsv0, tdv0, tsv1, tdv1, gsem0, gsem1):
        c = lax.axis_index("c")
        s = lax.axis_index("s")
        wid = s * NC + c
        base0 = wid * epw

        pltpu.sync_copy(src3_hbm.at[wid], srcall)
        pltpu.sync_copy(dst3_hbm.at[wid], dstall)

        def fire(j, tsv, tdv, sem):
            pltpu.async_copy(t_hbm.at[srcall.at[j]], tsv, sem)
            return pltpu.async_copy(t_hbm.at[dstall.at[j]], tdv, sem)

        def drain(tsv, sem):
            d = pltpu.make_async_copy(t_hbm.at[srcall.at[0]], tsv, sem)
            d.wait()
            d.wait()

        def writeback(j, tsv, tdv):
            pltpu.sync_copy(tsv, ts_out.at[pl.ds(base0 + j * gc, gc)])
            pltpu.sync_copy(tdv, td_out.at[pl.ds(base0 + j * gc, gc)])

        fire(0, tsv0, tdv0, gsem0)

        def pair(j2, carry):
            ja = 2 * j2
            jb = ja + 1
            jc = ja + 2
            fire(jb, tsv1, tdv1, gsem1)
            drain(tsv0, gsem0)
            writeback(ja, tsv0, tdv0)
            fire(jc, tsv0, tdv0, gsem0)
            drain(tsv1, gsem1)
            writeback(jb, tsv1, tdv1)
            return carry

        lax.fori_loop(0, npair, pair, 0)

        drain(tsv0, gsem0)
        writeback(n - 1, tsv0, tdv0)

    mesh = plsc.VectorSubcoreMesh(core_axis_name="c", subcore_axis_name="s")
    return pl.kernel(
        body,
        out_type=(
            jax.ShapeDtypeStruct((esl, TW), jnp.float32),
            jax.ShapeDtypeStruct((esl, TW), jnp.float32),
        ),
        mesh=mesh,
        scratch_types=[
            pltpu.VMEM((n, gc), jnp.int32),
            pltpu.VMEM((n, gc), jnp.int32),
            pltpu.VMEM((gc, TW), jnp.float32),
            pltpu.VMEM((gc, TW), jnp.float32),
            pltpu.VMEM((gc, TW), jnp.float32),
            pltpu.VMEM((gc, TW), jnp.float32),
            pltpu.SemaphoreType.DMA,
            pltpu.SemaphoreType.DMA,
        ],
    )


def _make_scatter(esl):
    ept = esl // NS
    gc = _chunk_of(ept)
    n = ept // gc
    npair = (n - 1) // 2

    def body(dh_hbm, dxp_hbm, dst3_hbm, zeros_hbm,
             aggh_out, aggx_out,
             dstall, rowv0, rowv1, acc, rsem0, rsem1):
        c = lax.axis_index("c")
        s = lax.axis_index("s")
        base0 = s * ept

        @pl.when(s == 0)
        def _():
            pltpu.sync_copy(zeros_hbm, acc)

        pltpu.sync_copy(dst3_hbm.at[s], dstall)
        plsc.subcore_barrier()

        def run(msg_hbm):
            def load(j, rv, sem):
                return pltpu.async_copy(
                    msg_hbm.at[pl.ds(base0 + j * gc, gc)], rv, sem)

            def drain0():
                pltpu.make_async_copy(
                    msg_hbm.at[pl.ds(base0, gc)], rowv0, rsem0).wait()

            load(0, rowv0, rsem0)

            def pair(j2, carry):
                ja = 2 * j2
                jb = ja + 1
                jc = ja + 2
                cb = load(jb, rowv1, rsem1)
                drain0()
                pltpu.sync_copy(rowv0, acc.at[dstall.at[ja]], add=True)
                load(jc, rowv0, rsem0)
                cb.wait()
                pltpu.sync_copy(rowv1, acc.at[dstall.at[jb]], add=True)
                return carry

            lax.fori_loop(0, npair, pair, 0)
            drain0()
            pltpu.sync_copy(rowv0, acc.at[dstall.at[n - 1]], add=True)

        @pl.when(c == 0)
        def _():
            run(dh_hbm)

        @pl.when(c == 1)
        def _():
            run(dxp_hbm)

        plsc.subcore_barrier()

        @pl.when((c == 0) & (s == 0))
        def _():
            pltpu.sync_copy(acc, aggh_out)

        @pl.when((c == 1) & (s == 0))
        def _():
            pltpu.sync_copy(acc, aggx_out)

    mesh = plsc.VectorSubcoreMesh(core_axis_name="c", subcore_axis_name="s")
    return pl.kernel(
        body,
        out_type=(
            jax.ShapeDtypeStruct((N, S), jnp.float32),
            jax.ShapeDtypeStruct((N, S), jnp.float32),
        ),
        mesh=mesh,
        scratch_types=[
            pltpu.VMEM((n, gc), jnp.int32),
            pltpu.VMEM((gc, S), jnp.float32),
            pltpu.VMEM((gc, S), jnp.float32),
            pltpu.VMEM_SHARED((N, S), jnp.float32),
            pltpu.SemaphoreType.DMA,
            pltpu.SemaphoreType.DMA,
        ],
    )


def _silu(z):
    return z * jax.nn.sigmoid(z)


def _mlp_body(ts, td, es, cent, wiv,
              w1a, w1b, w1c, w1d, b1, w2, b2, w3, b3,
              wg1, bg1, wg2, bg2, wh, bh, wx1, bx1, wx2, bx2,
              dh_out, dx_out):
    hs = ts[:, :S]
    hd = td[:, :S]
    r = ts[:, S:] - td[:, S:]
    d2 = jnp.sum(r * r, axis=1, keepdims=True)
    dist = jnp.sqrt(d2)
    t = (dist - cent[...]) * wiv[0, 0]
    rbf = jnp.exp(-(t * t))

    dot = functools.partial(jnp.dot, preferred_element_type=jnp.float32)
    m = (dot(hs, w1a[...]) + dot(hd, w1b[...])
         + dot(rbf, w1c[...]) + dot(es[...], w1d[...]) + b1[...])
    m = _silu(m)
    m = _silu(dot(m, w2[...]) + b2[...])
    m = _silu(dot(m, w3[...]) + b3[...])
    a = jax.nn.relu(dot(m, wg1[...]) + bg1[...])
    g = jax.nn.sigmoid(jnp.sum(a * wg2[...], axis=1, keepdims=True) + bg2[0, 0])
    m = m * g
    dh_out[...] = _silu(dot(m, wh[...]) + bh[...])
    cx = _silu(dot(m, wx1[...]) + bx1[...])
    coeff = (jnp.sum(cx * wx2[...], axis=1, keepdims=True) + bx2[0, 0]) * 0.08
    dx_out[...] = r * coeff


def _full(shape):
    return pl.BlockSpec(shape, lambda i: (0,) * len(shape))


def _mlp_call(ts, td, es, cent, wiv, wts):
    esl = ts.shape[0]
    in_specs = [
        pl.BlockSpec((BE, TW), lambda i: (i, 0)),
        pl.BlockSpec((BE, TW), lambda i: (i, 0)),
        pl.BlockSpec((BE, 16), lambda i: (i, 0)),
        _full(cent.shape), _full(wiv.shape),
    ] + [_full(w.shape) for w in wts]
    return pl.pallas_call(
        _mlp_body,
        grid=(esl // BE,),
        in_specs=in_specs,
        out_specs=[
            pl.BlockSpec((BE, S), lambda i: (i, 0)),
            pl.BlockSpec((BE, S), lambda i: (i, 0)),
        ],
        out_shape=[
            jax.ShapeDtypeStruct((esl, S), jnp.float32),
            jax.ShapeDtypeStruct((esl, S), jnp.float32),
        ],
    )(ts, td, es, cent, wiv, *wts)


def _fin_body(h, x16, lng, lnb, alpha, *refs):
    n_agg = (len(refs) - 2) // 2
    agghs = refs[:n_agg]
    aggxs = refs[n_agg:2 * n_agg]
    h_out, x_out = refs[2 * n_agg:]
    sa = jax.nn.sigmoid(alpha[0, 0])
    aggh = agghs[0][...]
    aggx = aggxs[0][...]
    for k in range(1, n_agg):
        aggh = aggh + agghs[k][...]
        aggx = aggx + aggxs[k][...]
    pre = h[...] + sa * aggh
    mu = jnp.mean(pre, axis=1, keepdims=True)
    cent = pre - mu
    var = jnp.mean(cent * cent, axis=1, keepdims=True)
    h_out[...] = cent * lax.rsqrt(var + 1e-5) * lng[...] + lnb[...]
    x_out[...] = x16[...] + aggx[:, :XP]


def _fin_call(h, x16, lng, lnb, alpha, agghs, aggxs):
    nsb = pl.BlockSpec((BN, S), lambda i: (i, 0))
    xsb = pl.BlockSpec((BN, XP), lambda i: (i, 0))
    return pl.pallas_call(
        _fin_body,
        grid=(N // BN,),
        in_specs=[
            nsb, xsb,
            _full(lng.shape), _full(lnb.shape), _full(alpha.shape),
        ] + [nsb] * (len(agghs) + len(aggxs)),
        out_specs=[nsb, xsb],
        out_shape=[
            jax.ShapeDtypeStruct((N, S), jnp.float32),
            jax.ShapeDtypeStruct((N, XP), jnp.float32),
        ],
    )(h, x16, lng, lnb, alpha, *agghs, *aggxs)


def kernel(h, x, edge_index, e_s, params, centers, widths):
    src = edge_index[0].astype(jnp.int32)
    dst = edge_index[1].astype(jnp.int32)
    xf = x.astype(jnp.float32)
    x16 = jnp.pad(xf, ((0, 0), (0, XP - 3)))
    T = jnp.concatenate([h, xf, jnp.zeros((N, TW - S - 3), jnp.float32)], axis=1)

    W1 = params['W1']
    w1a = W1[:, :S].T
    w1b = W1[:, S:2 * S].T
    w1c = jnp.pad(W1[:, 2 * S:2 * S + NRBF].T, ((0, RBP - NRBF), (0, 0)))
    w1d = W1[:, 2 * S + NRBF:].T
    b1 = params['b1'][None, :]
    w2 = params['W2'].T
    b2 = params['b2'][None, :]
    w3 = params['W3'].T
    b3 = params['b3'][None, :]
    wg1 = params['Wg1'].T
    bg1 = params['bg1'][None, :]
    wg2 = params['Wg2']
    bg2 = params['bg2'][None, :]
    wh = params['Wh'].T
    bh = params['bh'][None, :]
    wx1 = params['Wx1'].T
    bx1 = params['bx1'][None, :]
    wx2 = params['Wx2']
    bx2 = params['bx2'][None, :]
    cent = jnp.pad(centers[None, :], ((0, 0), (0, RBP - NRBF)))
    wiv = (1.0 / (widths + 1e-8)).reshape(1, 1)
    alpha = params['alpha'].reshape(1, 1)
    lng = params['ln_g'][None, :]
    lnb = params['ln_b'][None, :]

    wts = [w1a, w1b, w1c, w1d, b1, w2, b2, w3, b3,
           wg1, bg1, wg2, bg2, wh, bh, wx1, bx1, wx2, bx2]
    zeros = jnp.zeros((N, S), jnp.float32)

    esl = E // NSL
    epw = esl // NW
    gcg = _chunk_of(epw)
    ept = esl // NS
    gcs_ = _chunk_of(ept)
    gather_fn = _make_gather(esl)
    scatter_fn = _make_scatter(esl)
    agghs, aggxs = [], []
    for k in range(NSL):
        sl = slice(k * esl, (k + 1) * esl)
        src3 = src[sl].reshape(NW, epw // gcg, gcg)
        dst3 = dst[sl].reshape(NW, epw // gcg, gcg)
        dst3s = dst[sl].reshape(NS, ept // gcs_, gcs_)
        ts, td = gather_fn(T, src3, dst3)
        dh, dxp = _mlp_call(ts, td, e_s[sl], cent, wiv, wts)
        aggh_k, aggx_k = scatter_fn(dh, dxp, dst3s, zeros)
        agghs.append(aggh_k)
        aggxs.append(aggx_k)

    h_new, x_new16 = _fin_call(h, x16, lng, lnb, alpha, agghs, aggxs)
    return (h_new, x_new16[:, :3])


# speedup vs baseline: 1.4048x; 1.0333x over previous
import functools

import jax
import jax.numpy as jnp
from jax import lax
from jax.experimental import pallas as pl
from jax.experimental.pallas import tpu as pltpu
from jax.experimental.pallas import tpu_sc as plsc

N = 10000
E = 320000
S = 128
TW = 256
XP = 16
NRBF = 18
RBP = 32
NC = 2
NS = 16
NW = NC * NS
NSL = 2

BE = 3200
BN = 2000


def _chunk_of(n):
    for g in range(128, 0, -8):
        if n % g == 0 and (n // g) % 2 == 1:
            return g
    raise ValueError(n)


def _make_gather(esl):
    epw = esl // NW
    gc = _chunk_of(epw)
    n = epw // gc
    npair = (n - 1) // 2

    def body(t_hbm, src3_hbm, dst3_hbm, ts_out, td_out,
             srcall, dstall, tsv0, tdv0, tsv1, tdv1, gsem0, gsem1):
        c = lax.axis_index("c")
        s = lax.axis_index("s")
        wid = s * NC + c
        base0 = wid * epw

        pltpu.sync_copy(src3_hbm.at[wid], srcall)
        pltpu.sync_copy(dst3_hbm.at[wid], dstall)

        def fire(j, tsv, tdv, sem):
            pltpu.async_copy(t_hbm.at[srcall.at[j]], tsv, sem)
            return pltpu.async_copy(t_hbm.at[dstall.at[j]], tdv, sem)

        def drain(tsv, sem):
            d = pltpu.make_async_copy(t_hbm.at[srcall.at[0]], tsv, sem)
            d.wait()
            d.wait()

        def writeback(j, tsv, tdv):
            pltpu.sync_copy(tsv, ts_out.at[pl.ds(base0 + j * gc, gc)])
            pltpu.sync_copy(tdv, td_out.at[pl.ds(base0 + j * gc, gc)])

        fire(0, tsv0, tdv0, gsem0)

        def pair(j2, carry):
            ja = 2 * j2
            jb = ja + 1
            jc = ja + 2
            fire(jb, tsv1, tdv1, gsem1)
            drain(tsv0, gsem0)
            writeback(ja, tsv0, tdv0)
            fire(jc, tsv0, tdv0, gsem0)
            drain(tsv1, gsem1)
            writeback(jb, tsv1, tdv1)
            return carry

        lax.fori_loop(0, npair, pair, 0)

        drain(tsv0, gsem0)
        writeback(n - 1, tsv0, tdv0)

    mesh = plsc.VectorSubcoreMesh(core_axis_name="c", subcore_axis_name="s")
    return pl.kernel(
        body,
        out_type=(
            jax.ShapeDtypeStruct((esl, TW), jnp.float32),
            jax.ShapeDtypeStruct((esl, TW), jnp.float32),
        ),
        mesh=mesh,
        scratch_types=[
            pltpu.VMEM((n, gc), jnp.int32),
            pltpu.VMEM((n, gc), jnp.int32),
            pltpu.VMEM((gc, TW), jnp.float32),
            pltpu.VMEM((gc, TW), jnp.float32),
            pltpu.VMEM((gc, TW), jnp.float32),
            pltpu.VMEM((gc, TW), jnp.float32),
            pltpu.SemaphoreType.DMA,
            pltpu.SemaphoreType.DMA,
        ],
    )


def _make_scatter(esl):
    ept = esl // NS
    gc = _chunk_of(ept)
    n = ept // gc
    npair = (n - 1) // 2

    def body(dh_hbm, dxp_hbm, dst3_hbm, zeros_hbm,
             aggh_out, aggx_out,
             dstall, rowv0, rowv1, acc, rsem0, rsem1):
        c = lax.axis_index("c")
        s = lax.axis_index("s")
        base0 = s * ept

        @pl.when(s == 0)
        def _():
            pltpu.sync_copy(zeros_hbm, acc)

        pltpu.sync_copy(dst3_hbm.at[s], dstall)
        plsc.subcore_barrier()

        def run(msg_hbm):
            def load(j, rv, sem):
                return pltpu.async_copy(
                    msg_hbm.at[pl.ds(base0 + j * gc, gc)], rv, sem)

            def drain0():
                pltpu.make_async_copy(
                    msg_hbm.at[pl.ds(base0, gc)], rowv0, rsem0).wait()

            load(0, rowv0, rsem0)

            def pair(j2, carry):
                ja = 2 * j2
                jb = ja + 1
                jc = ja + 2
                cb = load(jb, rowv1, rsem1)
                drain0()
                pltpu.sync_copy(rowv0, acc.at[dstall.at[ja]], add=True)
                load(jc, rowv0, rsem0)
                cb.wait()
                pltpu.sync_copy(rowv1, acc.at[dstall.at[jb]], add=True)
                return carry

            lax.fori_loop(0, npair, pair, 0)
            drain0()
            pltpu.sync_copy(rowv0, acc.at[dstall.at[n - 1]], add=True)

        @pl.when(c == 0)
        def _():
            run(dh_hbm)

        @pl.when(c == 1)
        def _():
            run(dxp_hbm)

        plsc.subcore_barrier()

        @pl.when((c == 0) & (s == 0))
        def _():
            pltpu.sync_copy(acc, aggh_out)

        @pl.when((c == 1) & (s == 0))
        def _():
            pltpu.sync_copy(acc, aggx_out)

    mesh = plsc.VectorSubcoreMesh(core_axis_name="c", subcore_axis_name="s")
    return pl.kernel(
        body,
        out_type=(
            jax.ShapeDtypeStruct((N, S), jnp.float32),
            jax.ShapeDtypeStruct((N, S), jnp.float32),
        ),
        mesh=mesh,
        scratch_types=[
            pltpu.VMEM((n, gc), jnp.int32),
            pltpu.VMEM((gc, S), jnp.float32),
            pltpu.VMEM((gc, S), jnp.float32),
            pltpu.VMEM_SHARED((N, S), jnp.float32),
            pltpu.SemaphoreType.DMA,
            pltpu.SemaphoreType.DMA,
        ],
    )


def _silu(z):
    return z * jax.nn.sigmoid(z)


def _mlp_body(ts, td, es, cent, wiv,
              w1a, w1b, w1c, w1d, b1, w2, b2, w3, b3,
              wg1, bg1, wg2, bg2, wh, bh, wx1, bx1, wx2, bx2,
              dh_out, dx_out):
    hs = ts[:, :S]
    hd = td[:, :S]
    r = ts[:, S:] - td[:, S:]
    d2 = jnp.sum(r * r, axis=1, keepdims=True)
    dist = jnp.sqrt(d2)
    t = (dist - cent[...]) * wiv[0, 0]
    rbf = jnp.exp(-(t * t))

    dot = functools.partial(jnp.dot, preferred_element_type=jnp.float32)
    m = (dot(hs, w1a[...]) + dot(hd, w1b[...])
         + dot(rbf, w1c[...]) + dot(es[...], w1d[...]) + b1[...])
    m = _silu(m)
    m = _silu(dot(m, w2[...]) + b2[...])
    m = _silu(dot(m, w3[...]) + b3[...])
    a = jax.nn.relu(dot(m, wg1[...]) + bg1[...])
    g = jax.nn.sigmoid(jnp.sum(a * wg2[...], axis=1, keepdims=True) + bg2[0, 0])
    m = m * g
    dh_out[...] = _silu(dot(m, wh[...]) + bh[...])
    cx = _silu(dot(m, wx1[...]) + bx1[...])
    coeff = (jnp.sum(cx * wx2[...], axis=1, keepdims=True) + bx2[0, 0]) * 0.08
    dx_out[...] = r * coeff


def _full(shape):
    return pl.BlockSpec(shape, lambda i: (0,) * len(shape))


def _mlp_call(ts, td, es, cent, wiv, wts):
    esl = ts.shape[0]
    in_specs = [
        pl.BlockSpec((BE, TW), lambda i: (i, 0)),
        pl.BlockSpec((BE, TW), lambda i: (i, 0)),
        pl.BlockSpec((BE, 16), lambda i: (i, 0)),
        _full(cent.shape), _full(wiv.shape),
    ] + [_full(w.shape) for w in wts]
    return pl.pallas_call(
        _mlp_body,
        grid=(esl // BE,),
        in_specs=in_specs,
        out_specs=[
            pl.BlockSpec((BE, S), lambda i: (i, 0)),
            pl.BlockSpec((BE, S), lambda i: (i, 0)),
        ],
        out_shape=[
            jax.ShapeDtypeStruct((esl, S), jnp.float32),
            jax.ShapeDtypeStruct((esl, S), jnp.float32),
        ],
    )(ts, td, es, cent, wiv, *wts)


def _fin_body(h, x16, lng, lnb, alpha, *refs):
    n_agg = (len(refs) - 2) // 2
    agghs = refs[:n_agg]
    aggxs = refs[n_agg:2 * n_agg]
    h_out, x_out = refs[2 * n_agg:]
    sa = jax.nn.sigmoid(alpha[0, 0])
    aggh = agghs[0][...]
    aggx = aggxs[0][...]
    for k in range(1, n_agg):
        aggh = aggh + agghs[k][...]
        aggx = aggx + aggxs[k][...]
    pre = h[...] + sa * aggh
    mu = jnp.mean(pre, axis=1, keepdims=True)
    cent = pre - mu
    var = jnp.mean(cent * cent, axis=1, keepdims=True)
    h_out[...] = cent * lax.rsqrt(var + 1e-5) * lng[...] + lnb[...]
    x_out[...] = x16[...] + aggx[:, :XP]


def _fin_call(h, x16, lng, lnb, alpha, agghs, aggxs):
    nsb = pl.BlockSpec((BN, S), lambda i: (i, 0))
    xsb = pl.BlockSpec((BN, XP), lambda i: (i, 0))
    return pl.pallas_call(
        _fin_body,
        grid=(N // BN,),
        in_specs=[
            nsb, xsb,
            _full(lng.shape), _full(lnb.shape), _full(alpha.shape),
        ] + [nsb] * (len(agghs) + len(aggxs)),
        out_specs=[nsb, xsb],
        out_shape=[
            jax.ShapeDtypeStruct((N, S), jnp.float32),
            jax.ShapeDtypeStruct((N, XP), jnp.float32),
        ],
    )(h, x16, lng, lnb, alpha, *agghs, *aggxs)


def kernel(h, x, edge_index, e_s, params, centers, widths):
    src = edge_index[0].astype(jnp.int32)
    dst = edge_index[1].astype(jnp.int32)
    xf = x.astype(jnp.float32)
    x16 = jnp.pad(xf, ((0, 0), (0, XP - 3)))
    T = jnp.concatenate([h, xf, jnp.zeros((N, TW - S - 3), jnp.float32)], axis=1)

    W1 = params['W1']
    w1a = W1[:, :S].T
    w1b = W1[:, S:2 * S].T
    w1c = jnp.pad(W1[:, 2 * S:2 * S + NRBF].T, ((0, RBP - NRBF), (0, 0)))
    w1d = W1[:, 2 * S + NRBF:].T
    b1 = params['b1'][None, :]
    w2 = params['W2'].T
    b2 = params['b2'][None, :]
    w3 = params['W3'].T
    b3 = params['b3'][None, :]
    wg1 = params['Wg1'].T
    bg1 = params['bg1'][None, :]
    wg2 = params['Wg2']
    bg2 = params['bg2'][None, :]
    wh = params['Wh'].T
    bh = params['bh'][None, :]
    wx1 = params['Wx1'].T
    bx1 = params['bx1'][None, :]
    wx2 = params['Wx2']
    bx2 = params['bx2'][None, :]
    cent = jnp.pad(centers[None, :], ((0, 0), (0, RBP - NRBF)))
    wiv = (1.0 / (widths + 1e-8)).reshape(1, 1)
    alpha = params['alpha'].reshape(1, 1)
    lng = params['ln_g'][None, :]
    lnb = params['ln_b'][None, :]

    wts = [w1a, w1b, w1c, w1d, b1, w2, b2, w3, b3,
           wg1, bg1, wg2, bg2, wh, bh, wx1, bx1, wx2, bx2]
    zeros = jnp.zeros((N, S), jnp.float32)

    esl = E // NSL
    epw = esl // NW
    gcg = _chunk_of(epw)
    ept = esl // NS
    gcs_ = _chunk_of(ept)
    gather_fn = _make_gather(esl)
    scatter_fn = _make_scatter(esl)
    agghs, aggxs = [], []
    for k in range(NSL):
        sl = slice(k * esl, (k + 1) * esl)
        src3 = src[sl].reshape(NW, epw // gcg, gcg)
        dst3 = dst[sl].reshape(NW, epw // gcg, gcg)
        dst3s = dst[sl].reshape(NS, ept // gcs_, gcs_)
        ts, td = gather_fn(T, src3, dst3)
        dh, dxp = _mlp_call(ts, td, e_s[sl], cent, wiv, wts)
        aggh_k, aggx_k = scatter_fn(dh, dxp, dst3s, zeros)
        agghs.append(aggh_k)
        aggxs.append(aggx_k)

    h_new, x_new16 = _fin_call(h, x16, lng, lnb, alpha, agghs, aggxs)
    return (h_new, x_new16[:, :3])
